# baseline (speedup 1.0000x reference)
import jax
import jax.numpy as jnp
from jax import lax
from jax.experimental import pallas as pl
from jax.experimental.pallas import tpu as pltpu
from jax.experimental.pallas import tpu_sc as plsc

VOCAB = 1000000
EMBED = 64
MAX_SEQ = 200
BATCH = 4096

NUM_CORES = 2
NUM_SUBCORES = 16
NUM_WORKERS = NUM_CORES * NUM_SUBCORES
ROWS_PER_WORKER = BATCH // NUM_WORKERS
IDX_SPLIT = 2
IDX_CHUNK = MAX_SEQ // IDX_SPLIT
LANES = 16


def _body(x_hbm, we_hbm, pe_hbm, out_hbm, idx_v, rows_v, pe_v, sem):
    wid = lax.axis_index("s") * NUM_CORES + lax.axis_index("c")
    base = wid * ROWS_PER_WORKER

    pltpu.sync_copy(pe_hbm, pe_v)

    @pl.loop(0, ROWS_PER_WORKER)
    def _(i):
        b = base + i
        pltpu.sync_copy(x_hbm.at[b], idx_v)
        cp0 = pltpu.async_copy(
            we_hbm.at[idx_v.at[0]], rows_v.at[pl.ds(0, IDX_CHUNK)], sem
        )
        cp1 = pltpu.async_copy(
            we_hbm.at[idx_v.at[1]], rows_v.at[pl.ds(IDX_CHUNK, IDX_CHUNK)], sem
        )
        cp0.wait()
        cp1.wait()

        @pl.loop(0, MAX_SEQ)
        def _(r):
            for c in range(EMBED // LANES):
                sl = pl.ds(c * LANES, LANES)
                rows_v[r, sl] += pe_v[r, sl]

        pltpu.sync_copy(rows_v, out_hbm.at[b])


@jax.jit
def _emb_kernel(x2, we_table, pe_table):
    mesh = plsc.VectorSubcoreMesh(
        core_axis_name="c", subcore_axis_name="s", num_cores=NUM_CORES,
        num_subcores=NUM_SUBCORES,
    )
    return pl.kernel(
        _body,
        out_type=jax.ShapeDtypeStruct((BATCH, MAX_SEQ, EMBED), jnp.float32),
        mesh=mesh,
        scratch_types=[
            pltpu.VMEM((IDX_SPLIT, IDX_CHUNK), jnp.int32),
            pltpu.VMEM((MAX_SEQ, EMBED), jnp.float32),
            pltpu.VMEM((MAX_SEQ, EMBED), jnp.float32),
            pltpu.SemaphoreType.DMA,
        ],
        compiler_params=pltpu.CompilerParams(use_tc_tiling_on_sc=False),
    )(x2, we_table, pe_table)


def kernel(x, we_table, pe_table):
    x2 = x.reshape(BATCH, IDX_SPLIT, IDX_CHUNK)
    return _emb_kernel(x2, we_table, pe_table)


# speedup vs baseline: 1.2018x; 1.2018x over previous
import jax
import jax.numpy as jnp
from jax import lax
from jax.experimental import pallas as pl
from jax.experimental.pallas import tpu as pltpu
from jax.experimental.pallas import tpu_sc as plsc

VOCAB = 1000000
EMBED = 64
MAX_SEQ = 200
BATCH = 4096

NUM_CORES = 2
NUM_SUBCORES = 16
NUM_WORKERS = NUM_CORES * NUM_SUBCORES
ROWS_PER_WORKER = BATCH // NUM_WORKERS
PAIR = 2
PAIRS_PER_WORKER = ROWS_PER_WORKER // PAIR
CHUNK_SEQ = PAIR * MAX_SEQ
IDX_CHUNK = 100
GATHERS_PER_CHUNK = CHUNK_SEQ // IDX_CHUNK
LANES = 16


def _body(x_hbm, we_hbm, pe_hbm, out_hbm, idx_v, rows_v, pe_v, gsem, ssem):
    wid = lax.axis_index("s") * NUM_CORES + lax.axis_index("c")
    pair_base = wid * PAIRS_PER_WORKER

    pltpu.sync_copy(pe_hbm, pe_v)
    pltpu.sync_copy(x_hbm.at[wid], idx_v)

    def fire_gathers(slot, p):
        for j in range(GATHERS_PER_CHUNK):
            pltpu.async_copy(
                we_hbm.at[idx_v.at[p, j]],
                rows_v.at[slot, pl.ds(j * IDX_CHUNK, IDX_CHUNK)],
                gsem.at[slot],
            )

    def wait_gathers(slot):
        pltpu.make_async_copy(
            we_hbm.at[pl.ds(0, CHUNK_SEQ)], rows_v.at[slot], gsem.at[slot]
        ).wait()

    def fire_store(slot, p):
        pltpu.async_copy(
            rows_v.at[slot],
            out_hbm.at[pl.ds((pair_base + p) * CHUNK_SEQ, CHUNK_SEQ)],
            ssem.at[slot],
        )

    def wait_store(slot):
        pltpu.make_async_copy(
            we_hbm.at[pl.ds(0, CHUNK_SEQ)], rows_v.at[slot], ssem.at[slot]
        ).wait()

    def add_pe(slot):
        @pl.loop(0, MAX_SEQ)
        def _(r):
            for c in range(EMBED // LANES):
                sl = pl.ds(c * LANES, LANES)
                v = pe_v[r, sl]
                plsc.addupdate(rows_v.at[slot, r, sl], v)
                plsc.addupdate(rows_v.at[slot, MAX_SEQ + r, sl], v)

    fire_gathers(0, 0)

    @pl.loop(0, PAIRS_PER_WORKER, step=2)
    def _(p0):
        for q in range(2):
            p = p0 + q
            s = q
            o = 1 - q

            @pl.when(p >= 1)
            def _():
                wait_store(o)

            @pl.when(p + 1 < PAIRS_PER_WORKER)
            def _():
                fire_gathers(o, p + 1)

            wait_gathers(s)
            add_pe(s)
            fire_store(s, p)

    wait_store(1)


@jax.jit
def _emb_kernel(x4, we_table, pe_table):
    mesh = plsc.VectorSubcoreMesh(
        core_axis_name="c", subcore_axis_name="s", num_cores=NUM_CORES,
        num_subcores=NUM_SUBCORES,
    )
    return pl.kernel(
        _body,
        out_type=jax.ShapeDtypeStruct((BATCH * MAX_SEQ, EMBED), jnp.float32),
        mesh=mesh,
        scratch_types=[
            pltpu.VMEM(
                (PAIRS_PER_WORKER, GATHERS_PER_CHUNK, IDX_CHUNK), jnp.int32
            ),
            pltpu.VMEM((2, CHUNK_SEQ, EMBED), jnp.float32),
            pltpu.VMEM((MAX_SEQ, EMBED), jnp.float32),
            pltpu.SemaphoreType.DMA((2,)),
            pltpu.SemaphoreType.DMA((2,)),
        ],
        compiler_params=pltpu.CompilerParams(use_tc_tiling_on_sc=False),
    )(x4, we_table, pe_table)


def kernel(x, we_table, pe_table):
    x4 = x.reshape(NUM_WORKERS, PAIRS_PER_WORKER, GATHERS_PER_CHUNK, IDX_CHUNK)
    out = _emb_kernel(x4, we_table, pe_table)
    return out.reshape(BATCH, MAX_SEQ, EMBED)


# speedup vs baseline: 1.2051x; 1.0028x over previous
import jax
import jax.numpy as jnp
from jax import lax
from jax.experimental import pallas as pl
from jax.experimental.pallas import tpu as pltpu
from jax.experimental.pallas import tpu_sc as plsc

VOCAB = 1000000
EMBED = 64
MAX_SEQ = 200
BATCH = 4096

NUM_CORES = 2
NUM_SUBCORES = 16
NUM_WORKERS = NUM_CORES * NUM_SUBCORES
ROWS_PER_WORKER = BATCH // NUM_WORKERS
PAIR = 2
PAIRS_PER_WORKER = ROWS_PER_WORKER // PAIR
IDX_SPLITS = ((0, 128), (128, 72))
LANES = 16


def _body(x_hbm, we_hbm, pe_hbm, out_hbm, idx_v, rows_v, pe_v, gsem, ssem):
    wid = lax.axis_index("s") * NUM_CORES + lax.axis_index("c")
    row_base = wid * ROWS_PER_WORKER

    pltpu.sync_copy(pe_hbm, pe_v)
    pltpu.sync_copy(x_hbm.at[pl.ds(row_base, ROWS_PER_WORKER)], idx_v)

    def fire_gathers(slot, p):
        for k in range(PAIR):
            for off, ln in IDX_SPLITS:
                pltpu.async_copy(
                    we_hbm.at[idx_v.at[PAIR * p + k, pl.ds(off, ln)]],
                    rows_v.at[slot, k, pl.ds(off, ln)],
                    gsem.at[slot],
                )

    def wait_gathers(slot):
        for k in range(PAIR):
            pltpu.make_async_copy(
                we_hbm.at[pl.ds(0, MAX_SEQ)], rows_v.at[slot, k], gsem.at[slot]
            ).wait()

    def fire_store(slot, p):
        pltpu.async_copy(
            rows_v.at[slot],
            out_hbm.at[pl.ds(row_base + PAIR * p, PAIR)],
            ssem.at[slot],
        )

    def wait_store(slot):
        pltpu.make_async_copy(
            rows_v.at[slot], out_hbm.at[pl.ds(0, PAIR)], ssem.at[slot]
        ).wait()

    def add_pe(slot):
        @pl.loop(0, MAX_SEQ)
        def _(r):
            for c in range(EMBED // LANES):
                sl = pl.ds(c * LANES, LANES)
                v = pe_v[r, sl]
                plsc.addupdate(rows_v.at[slot, 0, r, sl], v)
                plsc.addupdate(rows_v.at[slot, 1, r, sl], v)

    fire_gathers(0, 0)

    @pl.loop(0, PAIRS_PER_WORKER, step=2)
    def _(p0):
        for q in range(2):
            p = p0 + q
            s = q
            o = 1 - q

            @pl.when(p >= 1)
            def _():
                wait_store(o)

            @pl.when(p + 1 < PAIRS_PER_WORKER)
            def _():
                fire_gathers(o, p + 1)

            wait_gathers(s)
            add_pe(s)
            fire_store(s, p)

    wait_store(1)


@jax.jit
def _emb_kernel(x, we_table, pe_table):
    mesh = plsc.VectorSubcoreMesh(
        core_axis_name="c", subcore_axis_name="s", num_cores=NUM_CORES,
        num_subcores=NUM_SUBCORES,
    )
    return pl.kernel(
        _body,
        out_type=jax.ShapeDtypeStruct((BATCH, MAX_SEQ, EMBED), jnp.float32),
        mesh=mesh,
        scratch_types=[
            pltpu.VMEM((ROWS_PER_WORKER, MAX_SEQ), jnp.int32),
            pltpu.VMEM((2, PAIR, MAX_SEQ, EMBED), jnp.float32),
            pltpu.VMEM((MAX_SEQ, EMBED), jnp.float32),
            pltpu.SemaphoreType.DMA((2,)),
            pltpu.SemaphoreType.DMA((2,)),
        ],
        compiler_params=pltpu.CompilerParams(use_tc_tiling_on_sc=False),
    )(x, we_table, pe_table)


def kernel(x, we_table, pe_table):
    return _emb_kernel(x, we_table, pe_table)
